# Initial kernel scaffold; baseline (speedup 1.0000x reference)
import jax
import jax.numpy as jnp
from jax.experimental import pallas as pl


def kernel(x, quantizer_emb):
    raise NotImplementedError("write your pallas kernel here")


# speedup vs baseline: 2.2521x; 2.2521x over previous
import jax
import jax.numpy as jnp
from jax.experimental import pallas as pl

_NQ = 8
_D = 256
_LT = 512


def _body(x_ref, emb_ref, o_ref):
    for qi in range(_NQ):
        o_ref[0, :, qi * _D:(qi + 1) * _D] = x_ref[0, qi] + emb_ref[qi]


def kernel(x, quantizer_emb):
    b, q, l, d = x.shape
    grid = (b, l // _LT)
    out = pl.pallas_call(
        _body,
        grid=grid,
        in_specs=[
            pl.BlockSpec((1, q, _LT, d), lambda i, j: (i, 0, j, 0)),
            pl.BlockSpec((q, d), lambda i, j: (0, 0)),
        ],
        out_specs=pl.BlockSpec((1, _LT, q * d), lambda i, j: (i, j, 0)),
        out_shape=jax.ShapeDtypeStruct((b, l, q * d), x.dtype),
    )(x, quantizer_emb)
    return out
